# Initial kernel scaffold; baseline (speedup 1.0000x reference)
import jax
import jax.numpy as jnp
from jax.experimental import pallas as pl


def kernel(gate_type, edge_index, edge_w, c_emb, c_w1_0, c_w2_0, c_b2_0, c_w1_r, c_w2_r, c_b2_r, a_emb, a_w1_0, a_w2_0, a_b2_0, a_w1_r, a_w2_r, a_b2_r, ch_w1, ch_b1, ch_w2, ch_b2, ah_w1, ah_b1, ah_w2, ah_b2):
    raise NotImplementedError("write your pallas kernel here")


# speedup vs baseline: 2.6453x; 2.6453x over previous
import functools

import jax
import jax.numpy as jnp
from jax import lax
from jax.experimental import pallas as pl
from jax.experimental.pallas import tpu as pltpu
from jax.experimental.pallas import tpu_sc as plsc

_N = 10000
_E = 320000
_GT = 26
_HID = 128
_NOUT = 64
_FH = 64
_KH = _FH // 16

_CH = 128
_TILES = 16
_EPAD = 327680
_CPT = _EPAD // (_TILES * _CH)
_HALFC = _CPT // 2
_NPAD = 10240
_RPT = _NPAD // _TILES
_RQ = 128

_NB = 5
_BLK = _N // _NB


def _make_edge_pass():
  mesh = plsc.VectorSubcoreMesh(core_axis_name="c", subcore_axis_name="s")
  out_type = [
      jax.ShapeDtypeStruct((2, _NPAD, _FH), jnp.float32),
      jax.ShapeDtypeStruct((_NPAD, 16), jnp.float32),
  ]
  scratch = [
      pltpu.VMEM((2, _HALFC, _CH), jnp.int32),
      pltpu.VMEM((3, _HALFC, _CH), jnp.float32),
      pltpu.VMEM((_CH, _FH), jnp.float32),
      pltpu.VMEM((_CH, _FH), jnp.float32),
      pltpu.VMEM((3, _FH), jnp.float32),
      pltpu.VMEM((_RQ, _FH), jnp.float32),
      pltpu.VMEM((_CH, 16), jnp.float32),
      pltpu.SemaphoreType.DMA,
      pltpu.SemaphoreType.DMA,
      pltpu.VMEM_SHARED((_NPAD, _FH), jnp.float32),
      pltpu.VMEM_SHARED((_NPAD, 16), jnp.float32),
  ]

  @functools.partial(pl.kernel, mesh=mesh, out_type=out_type,
                     scratch_types=scratch,
                     compiler_params=pltpu.CompilerParams(
                         use_tc_tiling_on_sc=False))
  def edge_pass(idx_hbm, ew_hbm, g_hbm, we_hbm, acc_out, cnt_out,
                idx_buf, ew_buf, rows0, rows1, w1e_buf, zbuf, ones_buf,
                gsem0, gsem1, acc_sh, cnt_sh):
    cid = lax.axis_index("c")
    sid = lax.axis_index("s")
    zv = jnp.zeros((16,), jnp.float32)

    def _zrow(r, _):
      for k in range(_KH):
        zbuf[r, pl.ds(k * 16, 16)] = zv
      return 0
    lax.fori_loop(0, _RQ, _zrow, 0)
    for q in range(_RPT // _RQ):
      pltpu.sync_copy(zbuf, acc_sh.at[pl.ds(sid * _RPT + q * _RQ, _RQ)])

    @pl.when(cid == 0)
    def _init_cnt():
      def _zrow16(r, _):
        ones_buf[r, :] = zv
        return 0
      lax.fori_loop(0, _CH, _zrow16, 0)
      for q in range(_RPT // _CH):
        pltpu.sync_copy(ones_buf,
                        cnt_sh.at[pl.ds(sid * _RPT + q * _CH, _CH)])
      ov = jnp.ones((16,), jnp.float32)
      def _orow(r, _):
        ones_buf[r, :] = ov
        return 0
      lax.fori_loop(0, _CH, _orow, 0)

    plsc.subcore_barrier()

    pltpu.sync_copy(we_hbm.at[cid], w1e_buf)
    wv = [[w1e_buf[j, pl.ds(k * 16, 16)] for k in range(_KH)]
          for j in range(3)]

    def _compute(rows, c):
      def _gbody(g, _):
        e0v = ew_buf[0, c, pl.ds(g * 16, 16)]
        e1v = ew_buf[1, c, pl.ds(g * 16, 16)]
        e2v = ew_buf[2, c, pl.ds(g * 16, 16)]
        for j in range(16):
          e = g * 16 + j
          e0, e1, e2 = e0v[j], e1v[j], e2v[j]
          for k in range(_KH):
            x = rows[e, pl.ds(k * 16, 16)]
            x = x + e0 * wv[0][k] + e1 * wv[1][k] + e2 * wv[2][k]
            rows[e, pl.ds(k * 16, 16)] = jnp.maximum(x, x * 0.01)
        return 0
      lax.fori_loop(0, _CH // 16, _gbody, 0)

    def _gather(c, rows, sem):
      pltpu.async_copy(g_hbm.at[cid].at[idx_buf.at[0, c]], rows, sem)

    def _gwait(c, rows, sem):
      pltpu.make_async_copy(g_hbm.at[cid].at[idx_buf.at[0, c]], rows,
                            sem).wait()

    def _scatter(rows, c):
      pltpu.sync_copy(rows, acc_sh.at[idx_buf.at[1, c]], add=True)

      @pl.when(cid == 0)
      def _():
        pltpu.sync_copy(ones_buf, cnt_sh.at[idx_buf.at[1, c]], add=True)

    for stage in range(2):
      hb = sid * _CPT + stage * _HALFC
      pltpu.sync_copy(idx_hbm.at[:, pl.ds(hb, _HALFC), :], idx_buf)
      pltpu.sync_copy(ew_hbm.at[:, pl.ds(hb, _HALFC), :], ew_buf)
      _gather(0, rows0, gsem0)

      def _it(i, _):
        c0 = 2 * i
        c1 = c0 + 1
        _gather(c1, rows1, gsem1)
        _gwait(c0, rows0, gsem0)
        _compute(rows0, c0)
        _scatter(rows0, c0)

        @pl.when(i < _HALFC // 2 - 1)
        def _():
          _gather(c0 + 2, rows0, gsem0)
        _gwait(c1, rows1, gsem1)
        _compute(rows1, c1)
        _scatter(rows1, c1)
        return 0
      lax.fori_loop(0, _HALFC // 2, _it, 0)

    plsc.subcore_barrier()

    for q in range(_RPT // _RQ):
      r0 = sid * _RPT + q * _RQ
      pltpu.sync_copy(acc_sh.at[pl.ds(r0, _RQ)], zbuf)
      pltpu.sync_copy(zbuf, acc_out.at[cid, pl.ds(r0, _RQ)])

    @pl.when(cid == 0)
    def _():
      for q in range(_RPT // _CH):
        r0 = sid * _RPT + q * _CH
        pltpu.sync_copy(cnt_sh.at[pl.ds(r0, _CH)], ones_buf)
        pltpu.sync_copy(ones_buf, cnt_out.at[pl.ds(r0, _CH)])

  return edge_pass


_edge_pass = _make_edge_pass()


def _edge_layer(idx3, ew3, g_halves, we2):
  acc0, cnt16 = _edge_pass(idx3, ew3, g_halves[0], we2[:, :, :_FH])
  acc1, _ = _edge_pass(idx3, ew3, g_halves[1], we2[:, :, _FH:])
  return acc0, acc1, cnt16


def _pair_spec(minor=_HID):
  return pl.BlockSpec((2, _BLK, minor), lambda i: (0, i, 0))


def _wfull(shape):
  return pl.BlockSpec(shape, lambda i: tuple(0 for _ in shape))


def _onehot(gt_blk):
  io = lax.broadcasted_iota(jnp.int32, (_BLK, 32), 1)
  return (io == gt_blk).astype(jnp.float32)


def _dot(a, b):
  return jnp.dot(a, b, preferred_element_type=jnp.float32)


def _g_shapes():
  return [jax.ShapeDtypeStruct((2, _N, _FH), jnp.float32)] * 2


def _init_g0_body(gt_ref, emb_ref, w_ref, g0_ref, g1_ref):
  oh = _onehot(gt_ref[...])
  for s in range(2):
    g = _dot(oh, _dot(emb_ref[s], w_ref[s]))
    g0_ref[s] = g[:, :_FH]
    g1_ref[s] = g[:, _FH:]


def _init_g0(gt2d, emb2, w10a2):
  return pl.pallas_call(
      _init_g0_body,
      grid=(_NB,),
      in_specs=[pl.BlockSpec((_BLK, 1), lambda i: (i, 0)),
                _wfull((2, 32, _GT)), _wfull((2, _GT, _HID))],
      out_specs=[_pair_spec(_FH), _pair_spec(_FH)],
      out_shape=_g_shapes(),
  )(gt2d, emb2, w10a2)


def _inv_cnt(cnt_blk):
  return 1.0 / jnp.maximum(cnt_blk[:, :1], 1.0)


def _combine0_body(gt_ref, acc0_ref, acc1_ref, cnt_ref, emb_ref, w20a_ref,
                   w20b_ref, b2_ref, w1n_ref, h_ref, g0_ref, g1_ref):
  oh = _onehot(gt_ref[...])
  inv = _inv_cnt(cnt_ref[...])
  for s in range(2):
    z = (_dot(oh, _dot(emb_ref[s], w20a_ref[s]))
         + _dot(acc0_ref[s] * inv, w20b_ref[s, :_FH])
         + _dot(acc1_ref[s] * inv, w20b_ref[s, _FH:]) + b2_ref[s])
    h = jnp.maximum(z, 0.0)
    h_ref[s] = h
    g = _dot(h, w1n_ref[s])
    g0_ref[s] = g[:, :_FH]
    g1_ref[s] = g[:, _FH:]


def _combine0(gt2d, acc0, acc1, cnt16, emb2, w20a2, w20b2, b202, w1n2):
  return pl.pallas_call(
      _combine0_body,
      grid=(_NB,),
      in_specs=[pl.BlockSpec((_BLK, 1), lambda i: (i, 0)),
                _pair_spec(_FH), _pair_spec(_FH),
                pl.BlockSpec((_BLK, 16), lambda i: (i, 0)),
                _wfull((2, 32, _GT)), _wfull((2, _GT, _HID)),
                _wfull((2, _HID, _HID)), _wfull((2, 1, _HID)),
                _wfull((2, _HID, _HID))],
      out_specs=[_pair_spec(), _pair_spec(_FH), _pair_spec(_FH)],
      out_shape=[jax.ShapeDtypeStruct((2, _N, _HID), jnp.float32)]
                + _g_shapes(),
  )(gt2d, acc0, acc1, cnt16, emb2, w20a2, w20b2, b202, w1n2)


def _make_combine(with_next):
  def body(h_ref, acc0_ref, acc1_ref, cnt_ref, w2a_ref, w2b_ref, b2_ref,
           *rest):
    if with_next:
      w1n_ref, ho_ref, g0_ref, g1_ref = rest
    else:
      (ho_ref,) = rest
    inv = _inv_cnt(cnt_ref[...])
    for s in range(2):
      z = (_dot(h_ref[s], w2a_ref[s])
           + _dot(acc0_ref[s] * inv, w2b_ref[s, :_FH])
           + _dot(acc1_ref[s] * inv, w2b_ref[s, _FH:]) + b2_ref[s])
      h = jnp.maximum(z, 0.0)
      ho_ref[s] = h
      if with_next:
        g = _dot(h, w1n_ref[s])
        g0_ref[s] = g[:, :_FH]
        g1_ref[s] = g[:, _FH:]

  def run(h2, acc0, acc1, cnt16, w2a2, w2b2, b22, *wn):
    in_specs = [_pair_spec(), _pair_spec(_FH), _pair_spec(_FH),
                pl.BlockSpec((_BLK, 16), lambda i: (i, 0)),
                _wfull((2, _HID, _HID)), _wfull((2, _HID, _HID)),
                _wfull((2, 1, _HID))]
    out_specs = [_pair_spec()]
    out_shape = [jax.ShapeDtypeStruct((2, _N, _HID), jnp.float32)]
    if with_next:
      in_specs.append(_wfull((2, _HID, _HID)))
      out_specs += [_pair_spec(_FH), _pair_spec(_FH)]
      out_shape += _g_shapes()
    return pl.pallas_call(
        body,
        grid=(_NB,),
        in_specs=in_specs,
        out_specs=out_specs,
        out_shape=out_shape,
    )(h2, acc0, acc1, cnt16, w2a2, w2b2, b22, *wn)

  return run


_combine_mid = _make_combine(True)
_combine_last = _make_combine(False)


def _heads1_body(h_ref, cw1_ref, cb1_ref, cw2_ref, cb2_ref,
                 aw1_ref, ab1_ref, aw2_ref, ab2_ref,
                 logits_ref, m_ref, s_ref, val_ref):
  i = pl.program_id(0)
  t = jnp.maximum(_dot(h_ref[0], cw1_ref[...]) + cb1_ref[...], 0.0)
  v = _dot(t, cw2_ref[...]) + cb2_ref[...]
  bsum = jnp.sum(v).reshape(1, 1)

  u = jnp.maximum(_dot(h_ref[1], aw1_ref[...]) + ab1_ref[...], 0.0)
  l = _dot(u, aw2_ref[...]) + ab2_ref[...]
  logits_ref[...] = l
  bm = jnp.max(l).reshape(1, 1)

  @pl.when(i == 0)
  def _():
    m_ref[...] = bm
    s_ref[...] = jnp.sum(jnp.exp(l - bm)).reshape(1, 1)
    val_ref[...] = bsum

  @pl.when(i > 0)
  def _():
    m_old = m_ref[...]
    s_old = s_ref[...]
    m_new = jnp.maximum(m_old, bm)
    m_ref[...] = m_new
    s_ref[...] = (s_old * jnp.exp(m_old - m_new)
                  + jnp.sum(jnp.exp(l - m_new)).reshape(1, 1))
    val_ref[...] = val_ref[...] + bsum

  @pl.when(i == _NB - 1)
  def _():
    val_ref[...] = val_ref[...] * (1.0 / _N)


def _heads1(h2, cw1, cb1, cw2, cb2, aw1, ab1, aw2, ab2):
  one = pl.BlockSpec((1, 1), lambda i: (0, 0))
  return pl.pallas_call(
      _heads1_body,
      grid=(_NB,),
      in_specs=[_pair_spec(),
                _wfull((_HID, _NOUT)), _wfull((1, _NOUT)),
                _wfull((_NOUT, 1)), _wfull((1, 1)),
                _wfull((_HID, _HID)), _wfull((1, _HID)),
                _wfull((_HID, _NOUT)), _wfull((1, _NOUT))],
      out_specs=[pl.BlockSpec((_BLK, _NOUT), lambda i: (i, 0)), one, one, one],
      out_shape=[jax.ShapeDtypeStruct((_N, _NOUT), jnp.float32),
                 jax.ShapeDtypeStruct((1, 1), jnp.float32),
                 jax.ShapeDtypeStruct((1, 1), jnp.float32),
                 jax.ShapeDtypeStruct((1, 1), jnp.float32)],
  )(h2, cw1, cb1, cw2, cb2, aw1, ab1, aw2, ab2)


def _heads2_body(l_ref, m_ref, s_ref, p_ref):
  p_ref[...] = jnp.exp(l_ref[...] - m_ref[...]) * (1.0 / s_ref[...])


def _heads2(logits, m, s):
  return pl.pallas_call(
      _heads2_body,
      grid=(_NB,),
      in_specs=[pl.BlockSpec((_BLK, _NOUT), lambda i: (i, 0)),
                pl.BlockSpec((1, 1), lambda i: (0, 0)),
                pl.BlockSpec((1, 1), lambda i: (0, 0))],
      out_specs=pl.BlockSpec((_BLK, _NOUT), lambda i: (i, 0)),
      out_shape=jax.ShapeDtypeStruct((_N, _NOUT), jnp.float32),
  )(logits, m, s)


def kernel(gate_type, edge_index, edge_w, c_emb, c_w1_0, c_w2_0, c_b2_0,
           c_w1_r, c_w2_r, c_b2_r, a_emb, a_w1_0, a_w2_0, a_b2_0, a_w1_r,
           a_w2_r, a_b2_r, ch_w1, ch_b1, ch_w2, ch_b2, ah_w1, ah_b1, ah_w2,
           ah_b2):
  f32 = jnp.float32
  i32 = jnp.int32
  npad = _EPAD - _E

  gt2d = gate_type.astype(i32).reshape(_N, 1)
  src_p = jnp.concatenate([edge_index[0].astype(i32),
                           jnp.zeros((npad,), i32)])
  dst_p = jnp.concatenate([edge_index[1].astype(i32),
                           jnp.full((npad,), _NPAD - 1, i32)])
  idx3 = jnp.stack([src_p, dst_p]).reshape(2, _EPAD // _CH, _CH)
  ew3 = jnp.concatenate([edge_w, jnp.zeros((npad, 3), f32)]
                        ).T.reshape(3, _EPAD // _CH, _CH)

  emb2 = jnp.stack([jnp.pad(c_emb, ((0, 32 - _GT), (0, 0))),
                    jnp.pad(a_emb, ((0, 32 - _GT), (0, 0)))])
  w10a2 = jnp.stack([c_w1_0[:_GT], a_w1_0[:_GT]])
  we0 = jnp.stack([c_w1_0[_GT:], a_w1_0[_GT:]])
  w20a2 = jnp.stack([c_w2_0[:_GT], a_w2_0[:_GT]])
  w20b2 = jnp.stack([c_w2_0[_GT:], a_w2_0[_GT:]])
  b202 = jnp.stack([c_b2_0.reshape(1, _HID), a_b2_0.reshape(1, _HID)])
  w1h = [jnp.stack([c_w1_r[j, :_HID], a_w1_r[j, :_HID]]) for j in range(4)]
  w1e = [jnp.stack([c_w1_r[j, _HID:], a_w1_r[j, _HID:]]) for j in range(4)]
  w2a = [jnp.stack([c_w2_r[j, :_HID], a_w2_r[j, :_HID]]) for j in range(4)]
  w2b = [jnp.stack([c_w2_r[j, _HID:], a_w2_r[j, _HID:]]) for j in range(4)]
  b2r = [jnp.stack([c_b2_r[j].reshape(1, _HID), a_b2_r[j].reshape(1, _HID)])
         for j in range(4)]

  g_halves = _init_g0(gt2d, emb2, w10a2)
  acc0, acc1, cnt16 = _edge_layer(idx3, ew3, g_halves, we0)
  h2, gh0, gh1 = _combine0(gt2d, acc0, acc1, cnt16, emb2, w20a2, w20b2,
                           b202, w1h[0])
  g_halves = (gh0, gh1)

  for j in range(4):
    acc0, acc1, _ = _edge_layer(idx3, ew3, g_halves, w1e[j])
    args = (h2, acc0, acc1, cnt16, w2a[j], w2b[j], b2r[j])
    if j < 3:
      h2, gh0, gh1 = _combine_mid(*args, w1h[j + 1])
      g_halves = (gh0, gh1)
    else:
      (h2,) = _combine_last(*args)

  logits, m, s, val = _heads1(
      h2, ch_w1, ch_b1.reshape(1, _NOUT), ch_w2, ch_b2.reshape(1, 1),
      ah_w1, ah_b1.reshape(1, _HID), ah_w2, ah_b2.reshape(1, _NOUT))
  probs = _heads2(logits, m, s).reshape(-1)
  value = val.reshape(())
  return (probs, value)
